# Initial kernel scaffold; baseline (speedup 1.0000x reference)
import jax
import jax.numpy as jnp
from jax.experimental import pallas as pl


def kernel(embed_t, h_t, context, W_x, W_h, W_ctx, b_ctx):
    raise NotImplementedError("write your pallas kernel here")


# speedup vs baseline: 1.1807x; 1.1807x over previous
import functools
import jax
import jax.numpy as jnp
from jax.experimental import pallas as pl
from jax.experimental.pallas import tpu as pltpu

B = 16384
EMBED = 512
HIDDEN = 1024
CTX = 1024

BLK = 1024


def _gate_body(e_ref, h_ref, c_ref, wx_ref, wh_ref, wc_ref, b_ref, o_ref):
    score = jnp.sum(e_ref[...] * wx_ref[...], axis=1)
    score += jnp.sum(h_ref[...] * wh_ref[...], axis=1)
    score += jnp.sum(c_ref[...] * wc_ref[...], axis=1)
    score += b_ref[0]
    o_ref[...] = jax.nn.sigmoid(score)


@jax.jit
def kernel(embed_t, h_t, context, W_x, W_h, W_ctx, b_ctx):
    grid = (B // BLK,)
    out = pl.pallas_call(
        _gate_body,
        grid=grid,
        in_specs=[
            pl.BlockSpec((BLK, EMBED), lambda i: (i, 0)),
            pl.BlockSpec((BLK, HIDDEN), lambda i: (i, 0)),
            pl.BlockSpec((BLK, CTX), lambda i: (i, 0)),
            pl.BlockSpec((1, EMBED), lambda i: (0, 0)),
            pl.BlockSpec((1, HIDDEN), lambda i: (0, 0)),
            pl.BlockSpec((1, CTX), lambda i: (0, 0)),
            pl.BlockSpec(memory_space=pltpu.SMEM),
        ],
        out_specs=pl.BlockSpec((BLK,), lambda i: (i,)),
        out_shape=jax.ShapeDtypeStruct((B,), jnp.float32),
    )(embed_t, h_t, context, W_x, W_h, W_ctx, b_ctx)
    return out
